# baseline (speedup 1.0000x reference)
import functools

import jax
import jax.numpy as jnp
from jax import lax
from jax.experimental import pallas as pl
from jax.experimental.pallas import tpu as pltpu
from jax.experimental.pallas import tpu_sc as plsc

N_EDGES = 320000
N_SEG = 10000
D = 128

NC = 2
NS = 16
NW = NC * NS
E_PER_TILE = N_EDGES // NW
CHUNK = 40
NCHUNK = E_PER_TILE // CHUNK
EXP_STRIDE = 624
EXP_ROWS = 640


def _sc_body(xr, idsr, zeros, out, pooled, idxb, vb, sems):
    c = lax.axis_index("c")
    s = lax.axis_index("s")
    wid = s * NC + c

    pltpu.sync_copy(zeros, pooled.at[pl.ds(s * EXP_STRIDE, EXP_ROWS)])
    pltpu.sync_copy(idsr.at[wid], idxb)
    plsc.subcore_barrier()

    pltpu.async_copy(xr.at[wid, 0], vb.at[0], sems.at[0])

    def step(j, carry):
        slot = lax.rem(j, 2)
        nslot = lax.rem(j + 1, 2)

        @pl.when(j + 1 < NCHUNK)
        def _prefetch():
            pltpu.async_copy(xr.at[wid, j + 1], vb.at[nslot], sems.at[nslot])

        pltpu.make_async_copy(xr.at[wid, j], vb.at[slot], sems.at[slot]).wait()
        pltpu.sync_copy(vb.at[slot], pooled.at[idxb.at[j]], add=True)
        return carry

    lax.fori_loop(0, NCHUNK, step, 0)
    plsc.subcore_barrier()

    sl = pl.ds(s * EXP_STRIDE, EXP_ROWS)
    pltpu.sync_copy(pooled.at[sl], out.at[c, sl])


_sc_segsum = functools.partial(
    pl.kernel,
    out_type=jax.ShapeDtypeStruct((NC, N_SEG, D), jnp.float32),
    mesh=plsc.VectorSubcoreMesh(
        core_axis_name="c", subcore_axis_name="s", num_cores=NC,
        num_subcores=NS),
    scratch_types=[
        pltpu.VMEM_SHARED((N_SEG, D), jnp.float32),
        pltpu.VMEM((NCHUNK, CHUNK), jnp.int32),
        pltpu.VMEM((2, CHUNK, D), jnp.float32),
        pltpu.SemaphoreType.DMA((2,)),
    ],
)(_sc_body)


def _tc_body(p_ref, wt_ref, b_ref, o_ref):
    p = p_ref[0] + p_ref[1]
    o_ref[...] = (
        jnp.dot(p, wt_ref[...], preferred_element_type=jnp.float32)
        + b_ref[...]
    )


def _tc_linear(partials, wt, b2):
    return pl.pallas_call(
        _tc_body,
        out_shape=jax.ShapeDtypeStruct((N_SEG, D), jnp.float32),
    )(partials, wt, b2)


def kernel(x, batch, W, b):
    ids = batch.astype(jnp.int32).reshape(NW, NCHUNK, CHUNK)
    xr = x.reshape(NW, NCHUNK, CHUNK, D)
    zeros = jnp.zeros((EXP_ROWS, D), jnp.float32)
    partials = _sc_segsum(xr, ids, zeros)
    return _tc_linear(partials, W.T, b.reshape(1, D))


# speedup vs baseline: 1.3456x; 1.3456x over previous
import functools

import jax
import jax.numpy as jnp
from jax import lax
from jax.experimental import pallas as pl
from jax.experimental.pallas import tpu as pltpu
from jax.experimental.pallas import tpu_sc as plsc

N_EDGES = 320000
N_SEG = 10000
D = 128

NC = 2
NS = 16
NW = NC * NS
E_PER_TILE = N_EDGES // NW
CH = 128
NFULL = E_PER_TILE // CH
TAIL = E_PER_TILE - NFULL * CH
NCH = NFULL + 1
POOL_ROWS = N_SEG + 16
EXP_STRIDE = 624
EXP_ROWS = 640


def _sc_body(x, idsr, zeros, out, pooled, idxb, vb, ldsem, scsem):
    c = lax.axis_index("c")
    s = lax.axis_index("s")
    wid = s * NC + c
    ebase = wid * E_PER_TILE

    pltpu.sync_copy(zeros, pooled.at[pl.ds(s * EXP_STRIDE, EXP_ROWS)])
    pltpu.sync_copy(idsr.at[wid], idxb)
    plsc.subcore_barrier()

    def load_start(j, slot):
        @pl.when(j < NFULL)
        def _full():
            pltpu.async_copy(
                x.at[pl.ds(ebase + j * CH, CH)], vb.at[slot], ldsem.at[slot])

        @pl.when(j == NFULL)
        def _tail():
            pltpu.async_copy(
                x.at[pl.ds(ebase + j * CH, TAIL)],
                vb.at[slot, pl.ds(0, TAIL)], ldsem.at[slot])

    def load_wait(j, slot):
        @pl.when(j < NFULL)
        def _full():
            pltpu.make_async_copy(
                x.at[pl.ds(ebase + j * CH, CH)], vb.at[slot],
                ldsem.at[slot]).wait()

        @pl.when(j == NFULL)
        def _tail():
            pltpu.make_async_copy(
                x.at[pl.ds(ebase + j * CH, TAIL)],
                vb.at[slot, pl.ds(0, TAIL)], ldsem.at[slot]).wait()

    def scatter_start(j, slot):
        pltpu.async_copy(
            vb.at[slot], pooled.at[idxb.at[j]], scsem.at[slot], add=True)

    def scatter_wait(j, slot):
        pltpu.make_async_copy(
            vb.at[slot], pooled.at[idxb.at[j]], scsem.at[slot]).wait()

    load_start(0, 0)

    def step(j, carry):
        slot = lax.rem(j, 2)
        nslot = lax.rem(j + 1, 2)

        @pl.when(j >= 1)
        def _retire():
            scatter_wait(j - 1, nslot)

        @pl.when(j + 1 < NCH)
        def _prefetch():
            load_start(j + 1, nslot)

        load_wait(j, slot)
        scatter_start(j, slot)
        return carry

    lax.fori_loop(0, NCH, step, 0)
    scatter_wait(NCH - 1, lax.rem(NCH - 1, 2))
    plsc.subcore_barrier()

    sl = pl.ds(s * EXP_STRIDE, EXP_ROWS)
    pltpu.sync_copy(pooled.at[sl], out.at[c, sl])


_sc_segsum = functools.partial(
    pl.kernel,
    out_type=jax.ShapeDtypeStruct((NC, N_SEG, D), jnp.float32),
    mesh=plsc.VectorSubcoreMesh(
        core_axis_name="c", subcore_axis_name="s", num_cores=NC,
        num_subcores=NS),
    scratch_types=[
        pltpu.VMEM_SHARED((POOL_ROWS, D), jnp.float32),
        pltpu.VMEM((NCH, CH), jnp.int32),
        pltpu.VMEM((2, CH, D), jnp.float32),
        pltpu.SemaphoreType.DMA((2,)),
        pltpu.SemaphoreType.DMA((2,)),
    ],
)(_sc_body)


def _tc_body(p_ref, wt_ref, b_ref, o_ref):
    p = p_ref[0] + p_ref[1]
    o_ref[...] = (
        jnp.dot(p, wt_ref[...], preferred_element_type=jnp.float32)
        + b_ref[...]
    )


def _tc_linear(partials, wt, b2):
    return pl.pallas_call(
        _tc_body,
        out_shape=jax.ShapeDtypeStruct((N_SEG, D), jnp.float32),
    )(partials, wt, b2)


def kernel(x, batch, W, b):
    ids = batch.astype(jnp.int32).reshape(NW, E_PER_TILE)
    pad = jnp.broadcast_to(
        N_SEG + (jnp.arange(NW, dtype=jnp.int32) % 16)[:, None],
        (NW, NCH * CH - E_PER_TILE))
    ids = jnp.concatenate([ids, pad], axis=1).reshape(NW, NCH, CH)
    zeros = jnp.zeros((EXP_ROWS, D), jnp.float32)
    partials = _sc_segsum(x, ids, zeros)
    return _tc_linear(partials, W.T, b.reshape(1, D))
